# Initial kernel scaffold; baseline (speedup 1.0000x reference)
import jax
import jax.numpy as jnp
from jax.experimental import pallas as pl


def kernel(inputs, loc, untransformed_scale):
    raise NotImplementedError("write your pallas kernel here")


# speedup vs baseline: 1.3074x; 1.3074x over previous
import functools

import jax
import jax.numpy as jnp
from jax import lax
from jax.experimental import pallas as pl
from jax.experimental.pallas import tpu as pltpu
from jax.experimental.pallas import tpu_sc as plsc

_D = 32
_B = 16384 * 20
_NC = 2
_NS = 16
_NW = _NC * _NS
_BPW = _B // _NW
_C = 1024
_NCHUNK = _BPW // _C


def _softplus16(x):
    t = jnp.exp(-jnp.abs(x))
    s = t / (t + 2.0)
    s2 = s * s
    p = s * (2.0 + s2 * (0.6666667 + s2 * (0.4 + s2 * 0.2857143)))
    return jnp.maximum(x, 0.0) + p


def _make_kernel():
    mesh = plsc.VectorSubcoreMesh(core_axis_name="c", subcore_axis_name="s")

    @functools.partial(
        pl.kernel,
        mesh=mesh,
        compiler_params=pltpu.CompilerParams(use_tc_tiling_on_sc=False),
        out_type=(
            jax.ShapeDtypeStruct((_B, _D), jnp.float32),
            jax.ShapeDtypeStruct((_B, _D), jnp.float32),
        ),
        scratch_types=[
            pltpu.VMEM((_C,), jnp.int32),
            pltpu.VMEM((_C, _D), jnp.float32),
            pltpu.VMEM((_C, _D), jnp.float32),
            pltpu.SemaphoreType.DMA,
            pltpu.SemaphoreType.DMA,
        ],
    )
    def gather_kernel(idx_hbm, loc_hbm, usc_hbm, out_loc, out_sc,
                      idx_v, loc_v, sc_v, sem_a, sem_b):
        wid = lax.axis_index("s") * _NC + lax.axis_index("c")
        base = wid * _BPW

        def chunk_body(ci, carry):
            off = base + ci * _C
            pltpu.sync_copy(idx_hbm.at[pl.ds(off, _C)], idx_v)
            cp_l = pltpu.async_copy(loc_hbm.at[idx_v], loc_v, sem_a)
            cp_s = pltpu.async_copy(usc_hbm.at[idx_v], sc_v, sem_b)
            cp_s.wait()

            def sp_body(i, c2):
                x0 = sc_v[i, pl.ds(0, 16)]
                x1 = sc_v[i, pl.ds(16, 16)]
                sc_v[i, pl.ds(0, 16)] = _softplus16(x0)
                sc_v[i, pl.ds(16, 16)] = _softplus16(x1)
                return c2

            lax.fori_loop(0, _C, sp_body, 0)
            pltpu.sync_copy(sc_v, out_sc.at[pl.ds(off, _C)])
            cp_l.wait()
            pltpu.sync_copy(loc_v, out_loc.at[pl.ds(off, _C)])
            return carry

        lax.fori_loop(0, _NCHUNK, chunk_body, 0)

    return gather_kernel


_GATHER = _make_kernel()


def kernel(inputs, loc, untransformed_scale):
    idx = inputs.astype(jnp.int32).reshape(-1)
    out_loc, out_sc = _GATHER(idx, loc, untransformed_scale)
    shp = inputs.shape + (_D,)
    return out_loc.reshape(shp), out_sc.reshape(shp)
